# baseline (speedup 1.0000x reference)
import functools

import jax
import jax.numpy as jnp
from jax import lax
from jax.experimental import pallas as pl
from jax.experimental.pallas import tpu as pltpu
from jax.experimental.pallas import tpu_sc as plsc

B = 4096
EMBED = 64
NCOMP = 16
NC = 2
NS = 16
L = 16
NW = NC * NS
BPW = B // NW
NG = BPW // L


def _sc_body(uid_hbm, iid_hbm, ut_hbm, it_hbm,
             gu0_hbm, gi0_hbm, gu1_hbm, gi1_hbm, gu2_hbm, gi2_hbm, w0_hbm,
             p0_hbm, d0_hbm, d1_hbm, d2_hbm,
             idx_u, idx_i, rows_u, rows_i,
             rg_u0, rg_i0, rg_u1, rg_i1, rg_u2, rg_i2,
             w0_v, p0_v, d0_v, d1_v, d2_v, sem):
    wid = lax.axis_index("s") * NC + lax.axis_index("c")
    base = wid * BPW

    pltpu.sync_copy(uid_hbm.at[pl.ds(base, BPW)], idx_u)
    pltpu.sync_copy(iid_hbm.at[pl.ds(base, BPW)], idx_i)
    pltpu.sync_copy(w0_hbm, w0_v)

    cps = [
        pltpu.async_copy(ut_hbm.at[idx_u], rows_u, sem),
        pltpu.async_copy(it_hbm.at[idx_i], rows_i, sem),
        pltpu.async_copy(gu0_hbm.at[idx_u], rg_u0, sem),
        pltpu.async_copy(gi0_hbm.at[idx_i], rg_i0, sem),
        pltpu.async_copy(gu1_hbm.at[idx_u], rg_u1, sem),
        pltpu.async_copy(gi1_hbm.at[idx_i], rg_i1, sem),
        pltpu.async_copy(gu2_hbm.at[idx_u], rg_u2, sem),
        pltpu.async_copy(gi2_hbm.at[idx_i], rg_i2, sem),
    ]
    for c in cps:
        c.wait()

    w0c = [w0_v[pl.ds(j * L, L)] for j in range(EMBED // L)]

    def group(g, carry):
        rows = g * L + lax.broadcasted_iota(jnp.int32, (L,), 0)
        acc = jnp.zeros((L,), jnp.float32)
        for e in range(EMBED):
            col = jnp.full((L,), e, jnp.int32)
            uv = plsc.load_gather(rows_u, [rows, col])
            iv = plsc.load_gather(rows_i, [rows, col])
            acc = acc + uv * iv * w0c[e // L][e % L]
        d0 = jnp.zeros((L,), jnp.float32)
        d1 = jnp.zeros((L,), jnp.float32)
        d2 = jnp.zeros((L,), jnp.float32)
        for c in range(NCOMP):
            col = jnp.full((L,), c, jnp.int32)
            d0 = d0 + plsc.load_gather(rg_u0, [rows, col]) * plsc.load_gather(rg_i0, [rows, col])
            d1 = d1 + plsc.load_gather(rg_u1, [rows, col]) * plsc.load_gather(rg_i1, [rows, col])
            d2 = d2 + plsc.load_gather(rg_u2, [rows, col]) * plsc.load_gather(rg_i2, [rows, col])
        p0_v[pl.ds(g * L, L)] = acc
        d0_v[pl.ds(g * L, L)] = d0
        d1_v[pl.ds(g * L, L)] = d1
        d2_v[pl.ds(g * L, L)] = d2
        return carry

    lax.fori_loop(0, NG, group, 0)

    pltpu.sync_copy(p0_v, p0_hbm.at[pl.ds(base, BPW)])
    pltpu.sync_copy(d0_v, d0_hbm.at[pl.ds(base, BPW)])
    pltpu.sync_copy(d1_v, d1_hbm.at[pl.ds(base, BPW)])
    pltpu.sync_copy(d2_v, d2_hbm.at[pl.ds(base, BPW)])


_sc_call = pl.kernel(
    _sc_body,
    out_type=(
        jax.ShapeDtypeStruct((B,), jnp.float32),
        jax.ShapeDtypeStruct((B,), jnp.float32),
        jax.ShapeDtypeStruct((B,), jnp.float32),
        jax.ShapeDtypeStruct((B,), jnp.float32),
    ),
    mesh=plsc.VectorSubcoreMesh(core_axis_name="c", subcore_axis_name="s"),
    compiler_params=pltpu.CompilerParams(
        needs_layout_passes=False, use_tc_tiling_on_sc=False),
    scratch_types=[
        pltpu.VMEM((BPW,), jnp.int32),
        pltpu.VMEM((BPW,), jnp.int32),
        pltpu.VMEM((BPW, EMBED), jnp.float32),
        pltpu.VMEM((BPW, EMBED), jnp.float32),
        pltpu.VMEM((BPW, NCOMP), jnp.float32),
        pltpu.VMEM((BPW, NCOMP), jnp.float32),
        pltpu.VMEM((BPW, NCOMP), jnp.float32),
        pltpu.VMEM((BPW, NCOMP), jnp.float32),
        pltpu.VMEM((BPW, NCOMP), jnp.float32),
        pltpu.VMEM((BPW, NCOMP), jnp.float32),
        pltpu.VMEM((EMBED,), jnp.float32),
        pltpu.VMEM((BPW,), jnp.float32),
        pltpu.VMEM((BPW,), jnp.float32),
        pltpu.VMEM((BPW,), jnp.float32),
        pltpu.VMEM((BPW,), jnp.float32),
        pltpu.SemaphoreType.DMA,
    ],
)


def _tail_body(p0_ref, d0_ref, d1_ref, d2_ref, w0_ref, w1_ref, w2_ref,
               b_ref, out_ref):
    w1 = w1_ref[...]
    w2 = w2_ref[...]
    b0 = b_ref[0, 0]
    b1 = b_ref[0, 1]
    b2 = b_ref[0, 2]
    t0 = jnp.sum(w0_ref[...])
    t1 = jnp.sum(w1)
    t2 = jnp.sum(w2)
    s1 = jnp.sum((p0_ref[...] + d0_ref[...] * t0) * w1) + b0 * t1
    r = jnp.sum(d1_ref[...] * w2)
    s2 = (s1 + b1) * t2 + t1 * r
    out_ref[...] = s2 + d2_ref[...] * t2 + b2


_tail_call = pl.pallas_call(
    _tail_body,
    out_shape=jax.ShapeDtypeStruct((32, 128), jnp.float32),
    in_specs=[pl.BlockSpec(memory_space=pltpu.VMEM)] * 7
    + [pl.BlockSpec(memory_space=pltpu.SMEM)],
    out_specs=pl.BlockSpec(memory_space=pltpu.VMEM),
)


def kernel(user_id, item_id, user_table, item_table,
           gcn_user_0, gcn_item_0, W_0, b_0,
           gcn_user_1, gcn_item_1, W_1, b_1,
           gcn_user_2, gcn_item_2, W_2, b_2):
    uid = user_id.reshape(B).astype(jnp.int32)
    iid = item_id.reshape(B).astype(jnp.int32)
    p0, d0, d1, d2 = _sc_call(
        uid, iid, user_table, item_table,
        gcn_user_0, gcn_item_0, gcn_user_1, gcn_item_1,
        gcn_user_2, gcn_item_2, W_0.reshape(EMBED))
    b = jnp.concatenate([b_0, b_1, b_2]).reshape(1, 3)
    out = _tail_call(
        p0.reshape(32, 128), d0.reshape(32, 128),
        d1.reshape(32, 128), d2.reshape(32, 128),
        W_0.reshape(1, EMBED),
        W_1.reshape(32, 128), W_2.reshape(32, 128), b)
    return out.reshape(B, 1)


# speedup vs baseline: 4.8913x; 4.8913x over previous
import jax
import jax.numpy as jnp
from jax import lax
from jax.experimental import pallas as pl
from jax.experimental.pallas import tpu as pltpu
from jax.experimental.pallas import tpu_sc as plsc

B = 4096
EMBED = 64
NCOMP = 16
NTAB = 100000
L = 16
NW = 32
NDIMS = 2 * EMBED + 6 * NCOMP
GSTEPS = B // L


def _gather_dim(src_t, e, idx_v, row_v, out_v, out_hbm, r_flat):
    pltpu.sync_copy(src_t.at[e, :], row_v)

    def gstep(j, carry):
        iv = idx_v[pl.ds(j * L, L)]
        out_v[pl.ds(j * L, L)] = plsc.load_gather(row_v, [iv])
        return carry

    lax.fori_loop(0, GSTEPS, gstep, 0)
    pltpu.sync_copy(out_v, out_hbm.at[pl.ds(r_flat * B, B)])


def _sc_body(uid_hbm, iid_hbm, ut_t, it_t,
             gu0_t, gi0_t, gu1_t, gi1_t, gu2_t, gi2_t,
             out_hbm,
             uid_v, iid_v, row_v, out_v):
    wid = lax.axis_index("s") * 2 + lax.axis_index("c")
    pltpu.sync_copy(uid_hbm, uid_v)
    pltpu.sync_copy(iid_hbm, iid_v)

    @pl.when(wid < 16)
    def _():
        for j in range(4):
            e = wid * 4 + j
            _gather_dim(ut_t, e, uid_v, row_v, out_v, out_hbm, e)

    @pl.when(wid >= 16)
    def _():
        for j in range(4):
            e = (wid - 16) * 4 + j
            _gather_dim(it_t, e, iid_v, row_v, out_v, out_hbm, EMBED + e)

    gtabs = [(gu0_t, uid_v), (gi0_t, iid_v),
             (gu1_t, uid_v), (gi1_t, iid_v),
             (gu2_t, uid_v), (gi2_t, iid_v)]
    for t, (tab, idxv) in enumerate(gtabs):
        lo_w = max(0, -(-(NCOMP * t - 2) // 3))
        hi_w = (NCOMP * t + NCOMP - 1) // 3

        @pl.when((wid >= lo_w) & (wid <= hi_w))
        def _(t=t, tab=tab, idxv=idxv):
            for j in range(3):
                g = 3 * wid + j

                @pl.when((g >= NCOMP * t) & (g < NCOMP * (t + 1)))
                def _(g=g, t=t, tab=tab, idxv=idxv):
                    e = g - NCOMP * t
                    _gather_dim(tab, e, idxv, row_v, out_v, out_hbm,
                                2 * EMBED + g)


_sc_call = pl.kernel(
    _sc_body,
    out_type=jax.ShapeDtypeStruct((NDIMS * B,), jnp.float32),
    mesh=plsc.VectorSubcoreMesh(core_axis_name="c", subcore_axis_name="s"),
    compiler_params=pltpu.CompilerParams(
        needs_layout_passes=False, use_tc_tiling_on_sc=True),
    scratch_types=[
        pltpu.VMEM((B,), jnp.int32),
        pltpu.VMEM((B,), jnp.int32),
        pltpu.VMEM((NTAB,), jnp.float32),
        pltpu.VMEM((B,), jnp.float32),
    ],
)


def _tail_body(g_ref, w0_ref, w1_ref, w2_ref, b_ref, out_ref):
    def dim(r):
        return g_ref[pl.ds(r * B, B)]

    p0 = jnp.zeros((B,), jnp.float32)
    for e in range(EMBED):
        p0 = p0 + dim(e) * dim(EMBED + e) * w0_ref[0, e]
    d0 = jnp.zeros((B,), jnp.float32)
    d1 = jnp.zeros((B,), jnp.float32)
    d2 = jnp.zeros((B,), jnp.float32)
    base = 2 * EMBED
    for c in range(NCOMP):
        d0 = d0 + dim(base + c) * dim(base + NCOMP + c)
        d1 = d1 + dim(base + 2 * NCOMP + c) * dim(base + 3 * NCOMP + c)
        d2 = d2 + dim(base + 4 * NCOMP + c) * dim(base + 5 * NCOMP + c)
    w1 = w1_ref[...]
    w2 = w2_ref[...]
    b0 = b_ref[0, 0]
    b1 = b_ref[0, 1]
    b2 = b_ref[0, 2]
    t0 = jnp.float32(0)
    for e in range(EMBED):
        t0 = t0 + w0_ref[0, e]
    t1 = jnp.sum(w1)
    t2 = jnp.sum(w2)
    out0 = p0 + d0 * t0 + b0
    s1 = jnp.sum(out0 * w1)
    r = jnp.sum(d1 * w2)
    s2 = (s1 + b1) * t2 + t1 * r
    out_ref[...] = s2 + d2 * t2 + b2


_tail_call = pl.pallas_call(
    _tail_body,
    out_shape=jax.ShapeDtypeStruct((B,), jnp.float32),
    in_specs=[
        pl.BlockSpec(memory_space=pltpu.VMEM),
        pl.BlockSpec(memory_space=pltpu.SMEM),
        pl.BlockSpec(memory_space=pltpu.VMEM),
        pl.BlockSpec(memory_space=pltpu.VMEM),
        pl.BlockSpec(memory_space=pltpu.SMEM),
    ],
    out_specs=pl.BlockSpec(memory_space=pltpu.VMEM),
)


def kernel(user_id, item_id, user_table, item_table,
           gcn_user_0, gcn_item_0, W_0, b_0,
           gcn_user_1, gcn_item_1, W_1, b_1,
           gcn_user_2, gcn_item_2, W_2, b_2):
    uid = user_id.reshape(B).astype(jnp.int32)
    iid = item_id.reshape(B).astype(jnp.int32)
    g = _sc_call(
        uid, iid, user_table.T, item_table.T,
        gcn_user_0.T, gcn_item_0.T, gcn_user_1.T, gcn_item_1.T,
        gcn_user_2.T, gcn_item_2.T)
    b = jnp.concatenate([b_0, b_1, b_2]).reshape(1, 3)
    out = _tail_call(g, W_0.reshape(1, EMBED), W_1.reshape(B),
                     W_2.reshape(B), b)
    return out.reshape(B, 1)
